# baseline (speedup 1.0000x reference)
import functools

import jax
import jax.numpy as jnp
from jax import lax
from jax.experimental import pallas as pl
from jax.experimental.pallas import tpu as pltpu
from jax.experimental.pallas import tpu_sc as plsc

_NC = 2
_NS = 16
_NW = _NC * _NS


def _sc_gather(u, i, P, Q, Pm, Qm, ub, ib):
    B = u.shape[0]
    bpw = B // _NW
    CH = 128
    nch = bpw // CH
    dm = P.shape[1]
    dmlp = Pm.shape[1]
    u3 = u.reshape(_NW, nch, CH)
    i3 = i.reshape(_NW, nch, CH)
    mesh = plsc.VectorSubcoreMesh(core_axis_name="c", subcore_axis_name="s")

    @functools.partial(
        pl.kernel,
        mesh=mesh,
        compiler_params=pltpu.CompilerParams(use_tc_tiling_on_sc=False),
        out_type=[
            jax.ShapeDtypeStruct((B, dm), jnp.float32),
            jax.ShapeDtypeStruct((B, dm), jnp.float32),
            jax.ShapeDtypeStruct((B, dmlp), jnp.float32),
            jax.ShapeDtypeStruct((B, dmlp), jnp.float32),
            jax.ShapeDtypeStruct((B, 1), jnp.float32),
            jax.ShapeDtypeStruct((B, 1), jnp.float32),
        ],
        scratch_types=[
            pltpu.VMEM((nch, CH), jnp.int32),
            pltpu.VMEM((nch, CH), jnp.int32),
            pltpu.VMEM((bpw, dm), jnp.float32),
            pltpu.VMEM((bpw, dm), jnp.float32),
            pltpu.VMEM((bpw, dmlp), jnp.float32),
            pltpu.VMEM((bpw, dmlp), jnp.float32),
            pltpu.VMEM((bpw, 1), jnp.float32),
            pltpu.VMEM((bpw, 1), jnp.float32),
            pltpu.SemaphoreType.DMA,
        ],
    )
    def k(u_hbm, i_hbm, p_hbm, q_hbm, pm_hbm, qm_hbm, ub_hbm, ib_hbm,
          pu_o, qi_o, pmu_o, qmi_o, ubu_o, ibi_o,
          uv, iv, puv, qiv, pmuv, qmiv, ubv, ibv, sem):
        wid = lax.axis_index("s") * _NC + lax.axis_index("c")
        base = wid * bpw
        pltpu.sync_copy(u_hbm.at[wid], uv)
        pltpu.sync_copy(i_hbm.at[wid], iv)
        copies = []
        for j in range(nch):
            sl = pl.ds(j * CH, CH)
            copies.append(pltpu.make_async_copy(p_hbm.at[uv.at[j]], puv.at[sl], sem))
            copies.append(pltpu.make_async_copy(q_hbm.at[iv.at[j]], qiv.at[sl], sem))
            copies.append(pltpu.make_async_copy(pm_hbm.at[uv.at[j]], pmuv.at[sl], sem))
            copies.append(pltpu.make_async_copy(qm_hbm.at[iv.at[j]], qmiv.at[sl], sem))
            copies.append(pltpu.make_async_copy(ub_hbm.at[uv.at[j]], ubv.at[sl], sem))
            copies.append(pltpu.make_async_copy(ib_hbm.at[iv.at[j]], ibv.at[sl], sem))
        for c in copies:
            c.start()
        for c in copies:
            c.wait()
        pltpu.sync_copy(puv, pu_o.at[pl.ds(base, bpw)])
        pltpu.sync_copy(qiv, qi_o.at[pl.ds(base, bpw)])
        pltpu.sync_copy(pmuv, pmu_o.at[pl.ds(base, bpw)])
        pltpu.sync_copy(qmiv, qmi_o.at[pl.ds(base, bpw)])
        pltpu.sync_copy(ubv, ubu_o.at[pl.ds(base, bpw)])
        pltpu.sync_copy(ibv, ibi_o.at[pl.ds(base, bpw)])

    return k(u3, i3, P, Q, Pm, Qm, ub, ib)


def _tc_mlp(pu, qi, pmu, qmi, ubu, ibi, W1, b1, W2, b2, Wo, bo):
    B = pu.shape[0]
    BLK = 2048
    dm = pu.shape[1]
    dmlp = pmu.shape[1]
    h1d = W1.shape[1]
    h2d = W2.shape[1]

    def body(pu_ref, qi_ref, pmu_ref, qmi_ref, ub_ref, ib_ref,
             w1_ref, b1_ref, w2_ref, b2_ref, wo_ref, bo_ref, o_ref):
        f32 = jnp.float32
        h1 = jnp.dot(pmu_ref[...], w1_ref[0:dmlp, :], preferred_element_type=f32)
        h1 += jnp.dot(qmi_ref[...], w1_ref[dmlp:2 * dmlp, :], preferred_element_type=f32)
        h1 = jnp.maximum(h1 + b1_ref[...], 0.0)
        h2 = jnp.maximum(
            jnp.dot(h1, w2_ref[...], preferred_element_type=f32) + b2_ref[...], 0.0)
        gmf = pu_ref[...] * qi_ref[...]
        s = jnp.dot(h2, wo_ref[0:h2d, :], preferred_element_type=f32)
        s += jnp.dot(gmf, wo_ref[h2d:h2d + dm, :], preferred_element_type=f32)
        o_ref[...] = s[:, 0] + bo_ref[0] + ub_ref[:, 0] + ib_ref[:, 0]

    grid = (B // BLK,)
    return pl.pallas_call(
        body,
        grid=grid,
        in_specs=[
            pl.BlockSpec((BLK, dm), lambda b: (b, 0)),
            pl.BlockSpec((BLK, dm), lambda b: (b, 0)),
            pl.BlockSpec((BLK, dmlp), lambda b: (b, 0)),
            pl.BlockSpec((BLK, dmlp), lambda b: (b, 0)),
            pl.BlockSpec((BLK, 1), lambda b: (b, 0)),
            pl.BlockSpec((BLK, 1), lambda b: (b, 0)),
            pl.BlockSpec((2 * dmlp, h1d), lambda b: (0, 0)),
            pl.BlockSpec((h1d,), lambda b: (0,)),
            pl.BlockSpec((h1d, h2d), lambda b: (0, 0)),
            pl.BlockSpec((h2d,), lambda b: (0,)),
            pl.BlockSpec((h2d + dm, 1), lambda b: (0, 0)),
            pl.BlockSpec((1,), lambda b: (0,)),
        ],
        out_specs=pl.BlockSpec((BLK,), lambda b: (b,)),
        out_shape=jax.ShapeDtypeStruct((B,), jnp.float32),
    )(pu, qi, pmu, qmi, ubu, ibi, W1, b1, W2, b2, Wo, bo)


def kernel(u, i, P, Q, Pm, Qm, W1, b1, W2, b2, Wo, bo, ub, ib):
    u = u.astype(jnp.int32)
    i = i.astype(jnp.int32)
    pu, qi, pmu, qmi, ubu, ibi = _sc_gather(u, i, P, Q, Pm, Qm, ub, ib)
    return _tc_mlp(pu, qi, pmu, qmi, ubu, ibi, W1, b1, W2, b2, Wo, bo)


# speedup vs baseline: 1.2850x; 1.2850x over previous
import dataclasses
import functools

import jax
import jax.numpy as jnp
from jax import lax
from jax.experimental import pallas as pl
from jax.experimental.pallas import tpu as pltpu
from jax.experimental.pallas import tpu_sc as plsc

_NC = 2
_NS = 16
_NW = _NC * _NS
_CH = 128
_LANES = 16


def _sc_compiler_params():
    cp = pltpu.CompilerParams(use_tc_tiling_on_sc=False)
    if "needs_layout_passes" in pltpu.CompilerParams.__dataclass_fields__:
        cp = dataclasses.replace(cp, needs_layout_passes=False)
    return cp


def _sc_gather(idx3, ridx3, PQ, CM, UBI):
    nw, nch, ch = idx3.shape
    bpw = nch * ch
    n2 = nw * bpw
    dpq = PQ.shape[1]
    dcm = CM.shape[1]
    mesh = plsc.VectorSubcoreMesh(core_axis_name="c", subcore_axis_name="s")

    @functools.partial(
        pl.kernel,
        mesh=mesh,
        compiler_params=_sc_compiler_params(),
        out_type=[
            jax.ShapeDtypeStruct((n2, dpq), jnp.float32),
            jax.ShapeDtypeStruct((n2, dcm), jnp.bfloat16),
            jax.ShapeDtypeStruct((n2,), jnp.float32),
        ],
        scratch_types=[
            pltpu.VMEM((nch, ch), jnp.int32),
            pltpu.VMEM((nch, ch), jnp.int32),
            pltpu.VMEM((bpw, dpq), jnp.float32),
            pltpu.VMEM((bpw, dcm), jnp.bfloat16),
            pltpu.VMEM((bpw, _LANES), jnp.float32),
            pltpu.VMEM((bpw,), jnp.float32),
            pltpu.SemaphoreType.DMA,
        ],
    )
    def k(idx_hbm, ridx_hbm, pq_hbm, cm_hbm, ubi_hbm,
          gpq_o, gcm_o, gb_o,
          iv, rv, pqv, cmv, ubiv, gbv, sem):
        wid = lax.axis_index("s") * _NC + lax.axis_index("c")
        base = wid * bpw
        pltpu.sync_copy(idx_hbm.at[wid], iv)
        pltpu.sync_copy(ridx_hbm.at[wid], rv)
        copies = []
        for j in range(nch):
            sl = pl.ds(j * ch, ch)
            copies.append(pltpu.make_async_copy(pq_hbm.at[iv.at[j]], pqv.at[sl], sem))
            copies.append(pltpu.make_async_copy(cm_hbm.at[iv.at[j]], cmv.at[sl], sem))
            copies.append(pltpu.make_async_copy(ubi_hbm.at[rv.at[j]], ubiv.at[sl], sem))
        for c in copies:
            c.start()
        for c in copies:
            c.wait()
        lane_iota = lax.iota(jnp.int32, _LANES)
        for j in range(nch):
            for c in range(ch // _LANES):
                idx16 = iv[j, pl.ds(c * _LANES, _LANES)]
                lanes = lax.rem(idx16, _LANES)
                rows = (j * ch + c * _LANES) + lane_iota
                gbv[pl.ds(j * ch + c * _LANES, _LANES)] = plsc.load_gather(
                    ubiv, [rows, lanes])
        pltpu.sync_copy(pqv, gpq_o.at[pl.ds(base, bpw)])
        pltpu.sync_copy(cmv, gcm_o.at[pl.ds(base, bpw)])
        pltpu.sync_copy(gbv, gb_o.at[pl.ds(base, bpw)])

    return k(idx3, ridx3, PQ, CM, UBI)


def _tc_mlp(gpq, gcm, gb, W1, b1, W2, b2, Wo, bo):
    n2 = gpq.shape[0]
    B = n2 // 2
    BLK = 2048
    noff = B // BLK
    dm = gpq.shape[1]
    dmlp = gcm.shape[1]
    h1d = W1.shape[1]
    h2d = W2.shape[1]

    def body(pu_ref, qi_ref, pmu_ref, qmi_ref, ub_ref, ib_ref,
             w1_ref, b1_ref, w2_ref, b2_ref, wo_ref, bo_ref, o_ref):
        f32 = jnp.float32
        bf16 = jnp.bfloat16
        h1 = jnp.dot(pmu_ref[...], w1_ref[0:dmlp, :].astype(bf16),
                     preferred_element_type=f32)
        h1 += jnp.dot(qmi_ref[...], w1_ref[dmlp:2 * dmlp, :].astype(bf16),
                      preferred_element_type=f32)
        h1 = jnp.maximum(h1 + b1_ref[...], 0.0)
        h2 = jnp.dot(h1.astype(bf16), w2_ref[...].astype(bf16),
                     preferred_element_type=f32) + b2_ref[...]
        h2 = jnp.maximum(h2, 0.0)
        gmf = pu_ref[...] * qi_ref[...]
        s = jnp.dot(h2, wo_ref[0:h2d, :], preferred_element_type=f32)
        s += jnp.dot(gmf, wo_ref[h2d:h2d + dm, :], preferred_element_type=f32)
        o_ref[...] = s[:, 0] + bo_ref[0] + ub_ref[...] + ib_ref[...]

    grid = (B // BLK,)
    return pl.pallas_call(
        body,
        grid=grid,
        in_specs=[
            pl.BlockSpec((BLK, dm), lambda b: (b, 0)),
            pl.BlockSpec((BLK, dm), lambda b: (b + noff, 0)),
            pl.BlockSpec((BLK, dmlp), lambda b: (b, 0)),
            pl.BlockSpec((BLK, dmlp), lambda b: (b + noff, 0)),
            pl.BlockSpec((BLK,), lambda b: (b,)),
            pl.BlockSpec((BLK,), lambda b: (b + noff,)),
            pl.BlockSpec((2 * dmlp, h1d), lambda b: (0, 0)),
            pl.BlockSpec((h1d,), lambda b: (0,)),
            pl.BlockSpec((h1d, h2d), lambda b: (0, 0)),
            pl.BlockSpec((h2d,), lambda b: (0,)),
            pl.BlockSpec((h2d + dm, 1), lambda b: (0, 0)),
            pl.BlockSpec((1,), lambda b: (0,)),
        ],
        out_specs=pl.BlockSpec((BLK,), lambda b: (b,)),
        out_shape=jax.ShapeDtypeStruct((B,), jnp.float32),
    )(gpq, gpq, gcm, gcm, gb, gb, W1, b1, W2, b2, Wo, bo)


def kernel(u, i, P, Q, Pm, Qm, W1, b1, W2, b2, Wo, bo, ub, ib):
    B = u.shape[0]
    n = P.shape[0]
    u = u.astype(jnp.int32)
    i = i.astype(jnp.int32)
    PQ = jnp.concatenate([P, Q], axis=0)
    CM = jnp.concatenate([Pm.astype(jnp.bfloat16), Qm.astype(jnp.bfloat16)], axis=0)
    UBI = jnp.concatenate([ub, ib], axis=0).reshape(-1, _LANES)
    idx = jnp.concatenate([u, i + n])
    bpw = (2 * B) // _NW
    idx3 = idx.reshape(_NW, bpw // _CH, _CH)
    ridx3 = idx3 // _LANES
    gpq, gcm, gb = _sc_gather(idx3, ridx3, PQ, CM, UBI)
    return _tc_mlp(gpq, gcm, gb, W1, b1, W2, b2, Wo, bo)


# speedup vs baseline: 1.2955x; 1.0082x over previous
import functools

import jax
import jax.numpy as jnp
from jax import lax
from jax.experimental import pallas as pl
from jax.experimental.pallas import tpu as pltpu
from jax.experimental.pallas import tpu_sc as plsc

_NC = 2
_NS = 16
_NW = _NC * _NS
_CH = 128
_W = 128


def _sc_gather(idx3, C):
    nw, nch, ch = idx3.shape
    bpw = nch * ch
    n2 = nw * bpw
    cpb = 2
    stage = cpb * ch
    nstg = nch // cpb
    mesh = plsc.VectorSubcoreMesh(core_axis_name="c", subcore_axis_name="s")

    @functools.partial(
        pl.kernel,
        mesh=mesh,
        out_type=jax.ShapeDtypeStruct((n2, _W), jnp.float32),
        scratch_types=[
            pltpu.VMEM((nch, ch), jnp.int32),
            pltpu.VMEM((stage, _W), jnp.float32),
            pltpu.VMEM((stage, _W), jnp.float32),
            pltpu.SemaphoreType.DMA,
            pltpu.SemaphoreType.DMA,
            pltpu.SemaphoreType.DMA,
            pltpu.SemaphoreType.DMA,
        ],
    )
    def k(idx_hbm, c_hbm, g_o, iv, rows0, rows1, gsem0, gsem1, osem0, osem1):
        wid = lax.axis_index("s") * _NC + lax.axis_index("c")
        base = wid * bpw
        pltpu.sync_copy(idx_hbm.at[wid], iv)
        bufs = (rows0, rows1)
        gsems = (gsem0, gsem1)
        osems = (osem0, osem1)

        def gathers(s):
            b = s % 2
            return [
                pltpu.make_async_copy(
                    c_hbm.at[iv.at[s * cpb + j]],
                    bufs[b].at[pl.ds(j * ch, ch)], gsems[b])
                for j in range(cpb)
            ]

        def copy_out(s):
            b = s % 2
            return pltpu.make_async_copy(
                bufs[b], g_o.at[pl.ds(base + s * stage, stage)], osems[b])

        for c in gathers(0):
            c.start()
        for s in range(nstg):
            for c in gathers(s):
                c.wait()
            if s + 1 < nstg:
                if s >= 1:
                    copy_out(s - 1).wait()
                for c in gathers(s + 1):
                    c.start()
            copy_out(s).start()
        copy_out(nstg - 1).wait()

    return k(idx3, C)


def _tc_mlp(G, W1, b1, W2, b2, Wo, bo):
    n2 = G.shape[0]
    B = n2 // 2
    BLK = 2048
    noff = B // BLK
    dmlp = 64
    dm = 32
    h1d = W1.shape[1]
    h2d = W2.shape[1]

    def body(gu_ref, gi_ref,
             w1_ref, b1_ref, w2_ref, b2_ref, wo_ref, bo_ref, o_ref):
        f32 = jnp.float32
        gu = gu_ref[...]
        gi = gi_ref[...]
        pmu = gu[:, 0:dmlp]
        qmi = gi[:, 0:dmlp]
        pu = gu[:, dmlp:dmlp + dm]
        qi = gi[:, dmlp:dmlp + dm]
        h1 = jnp.dot(pmu, w1_ref[0:dmlp, :], preferred_element_type=f32)
        h1 += jnp.dot(qmi, w1_ref[dmlp:2 * dmlp, :], preferred_element_type=f32)
        h1 = jnp.maximum(h1 + b1_ref[...], 0.0)
        h2 = jnp.dot(h1, w2_ref[...], preferred_element_type=f32) + b2_ref[...]
        h2 = jnp.maximum(h2, 0.0)
        gmf = pu * qi
        s = jnp.dot(h2, wo_ref[0:h2d, :], preferred_element_type=f32)
        s += jnp.dot(gmf, wo_ref[h2d:h2d + dm, :], preferred_element_type=f32)
        o_ref[...] = (s[:, 0] + bo_ref[0]
                      + gu[:, dmlp + dm] + gi[:, dmlp + dm])

    grid = (B // BLK,)
    return pl.pallas_call(
        body,
        grid=grid,
        in_specs=[
            pl.BlockSpec((BLK, _W), lambda b: (b, 0)),
            pl.BlockSpec((BLK, _W), lambda b: (b + noff, 0)),
            pl.BlockSpec((2 * dmlp, h1d), lambda b: (0, 0)),
            pl.BlockSpec((h1d,), lambda b: (0,)),
            pl.BlockSpec((h1d, h2d), lambda b: (0, 0)),
            pl.BlockSpec((h2d,), lambda b: (0,)),
            pl.BlockSpec((h2d + dm, 1), lambda b: (0, 0)),
            pl.BlockSpec((1,), lambda b: (0,)),
        ],
        out_specs=pl.BlockSpec((BLK,), lambda b: (b,)),
        out_shape=jax.ShapeDtypeStruct((B,), jnp.float32),
    )(G, G, W1, b1, W2, b2, Wo, bo)


def kernel(u, i, P, Q, Pm, Qm, W1, b1, W2, b2, Wo, bo, ub, ib):
    B = u.shape[0]
    n = P.shape[0]
    u = u.astype(jnp.int32)
    i = i.astype(jnp.int32)
    z = jnp.zeros((n, _W - Pm.shape[1] - P.shape[1] - 1), jnp.float32)
    C = jnp.concatenate(
        [jnp.concatenate([Pm, P, ub, z], axis=1),
         jnp.concatenate([Qm, Q, ib, z], axis=1)], axis=0)
    idx = jnp.concatenate([u, i + n])
    bpw = (2 * B) // _NW
    idx3 = idx.reshape(_NW, bpw // _CH, _CH)
    G = _sc_gather(idx3, C)
    return _tc_mlp(G, W1, b1, W2, b2, Wo, bo)


# speedup vs baseline: 4.6721x; 3.6065x over previous
import functools

import jax
import jax.numpy as jnp
from jax import lax
from jax.experimental import pallas as pl
from jax.experimental.pallas import tpu as pltpu
from jax.experimental.pallas import tpu_sc as plsc

_NC = 2
_NS = 16
_NW = _NC * _NS
_CH = 128
_W = 128
_RB = 4096


def _tc_pack(Et, Gt, bt):
    n = Et.shape[1]
    nblk = (n + _RB - 1) // _RB
    dmlp = Et.shape[0]
    dm = Gt.shape[0]
    dall = dmlp + dm + 1

    def body(e_ref, g_ref, b_ref, o_ref):
        sel = (lax.broadcasted_iota(jnp.int32, (dall, _W), 0)
               == lax.broadcasted_iota(jnp.int32, (dall, _W), 1)
               ).astype(jnp.float32)
        x = jnp.concatenate([e_ref[...], g_ref[...], b_ref[...]], axis=0)
        o_ref[...] = lax.dot_general(
            x, sel, (((0,), (0,)), ((), ())),
            preferred_element_type=jnp.float32)

    return pl.pallas_call(
        body,
        grid=(nblk,),
        in_specs=[
            pl.BlockSpec((dmlp, _RB), lambda b: (0, b)),
            pl.BlockSpec((dm, _RB), lambda b: (0, b)),
            pl.BlockSpec((1, _RB), lambda b: (0, b)),
        ],
        out_specs=pl.BlockSpec((_RB, _W), lambda b: (b, 0)),
        out_shape=jax.ShapeDtypeStruct((nblk * _RB, _W), jnp.float32),
    )(Et, Gt, bt)


def _sc_gather(idx3, C):
    nw, nch, ch = idx3.shape
    bpw = nch * ch
    n2 = nw * bpw
    cpb = 2
    stage = cpb * ch
    nstg = nch // cpb
    mesh = plsc.VectorSubcoreMesh(core_axis_name="c", subcore_axis_name="s")

    @functools.partial(
        pl.kernel,
        mesh=mesh,
        out_type=jax.ShapeDtypeStruct((n2, _W), jnp.float32),
        scratch_types=[
            pltpu.VMEM((nch, ch), jnp.int32),
            pltpu.VMEM((stage, _W), jnp.float32),
            pltpu.VMEM((stage, _W), jnp.float32),
            pltpu.SemaphoreType.DMA,
            pltpu.SemaphoreType.DMA,
            pltpu.SemaphoreType.DMA,
            pltpu.SemaphoreType.DMA,
        ],
    )
    def k(idx_hbm, c_hbm, g_o, iv, rows0, rows1, gsem0, gsem1, osem0, osem1):
        wid = lax.axis_index("s") * _NC + lax.axis_index("c")
        base = wid * bpw
        pltpu.sync_copy(idx_hbm.at[wid], iv)
        bufs = (rows0, rows1)
        gsems = (gsem0, gsem1)
        osems = (osem0, osem1)

        def gathers(s):
            b = s % 2
            return [
                pltpu.make_async_copy(
                    c_hbm.at[iv.at[s * cpb + j]],
                    bufs[b].at[pl.ds(j * ch, ch)], gsems[b])
                for j in range(cpb)
            ]

        def copy_out(s):
            b = s % 2
            return pltpu.make_async_copy(
                bufs[b], g_o.at[pl.ds(base + s * stage, stage)], osems[b])

        for c in gathers(0):
            c.start()
        for s in range(nstg):
            for c in gathers(s):
                c.wait()
            if s + 1 < nstg:
                if s >= 1:
                    copy_out(s - 1).wait()
                for c in gathers(s + 1):
                    c.start()
            copy_out(s).start()
        copy_out(nstg - 1).wait()

    return k(idx3, C)


def _tc_mlp(Gu, Gi, W1, b1, W2, b2, Wo, bo):
    B = Gu.shape[0]
    BLK = 2048
    dmlp = 64
    dm = 32
    h1d = W1.shape[1]
    h2d = W2.shape[1]

    def body(gu_ref, gi_ref,
             w1_ref, b1_ref, w2_ref, b2_ref, wo_ref, bo_ref, o_ref):
        f32 = jnp.float32
        gu = gu_ref[...]
        gi = gi_ref[...]
        pmu = gu[:, 0:dmlp]
        qmi = gi[:, 0:dmlp]
        pu = gu[:, dmlp:dmlp + dm]
        qi = gi[:, dmlp:dmlp + dm]
        h1 = jnp.dot(pmu, w1_ref[0:dmlp, :], preferred_element_type=f32)
        h1 += jnp.dot(qmi, w1_ref[dmlp:2 * dmlp, :], preferred_element_type=f32)
        h1 = jnp.maximum(h1 + b1_ref[...], 0.0)
        h2 = jnp.dot(h1, w2_ref[...], preferred_element_type=f32) + b2_ref[...]
        h2 = jnp.maximum(h2, 0.0)
        gmf = pu * qi
        s = jnp.dot(h2, wo_ref[0:h2d, :], preferred_element_type=f32)
        s += jnp.dot(gmf, wo_ref[h2d:h2d + dm, :], preferred_element_type=f32)
        o_ref[...] = (s[:, 0] + bo_ref[0]
                      + gu[:, dmlp + dm] + gi[:, dmlp + dm])

    grid = (B // BLK,)
    return pl.pallas_call(
        body,
        grid=grid,
        in_specs=[
            pl.BlockSpec((BLK, _W), lambda b: (b, 0)),
            pl.BlockSpec((BLK, _W), lambda b: (b, 0)),
            pl.BlockSpec((2 * dmlp, h1d), lambda b: (0, 0)),
            pl.BlockSpec((h1d,), lambda b: (0,)),
            pl.BlockSpec((h1d, h2d), lambda b: (0, 0)),
            pl.BlockSpec((h2d,), lambda b: (0,)),
            pl.BlockSpec((h2d + dm, 1), lambda b: (0, 0)),
            pl.BlockSpec((1,), lambda b: (0,)),
        ],
        out_specs=pl.BlockSpec((BLK,), lambda b: (b,)),
        out_shape=jax.ShapeDtypeStruct((B,), jnp.float32),
    )(Gu, Gi, W1, b1, W2, b2, Wo, bo)


def kernel(u, i, P, Q, Pm, Qm, W1, b1, W2, b2, Wo, bo, ub, ib):
    B = u.shape[0]
    u = u.astype(jnp.int32)
    i = i.astype(jnp.int32)
    Cu = _tc_pack(Pm.T, P.T, ub.T)
    Ci = _tc_pack(Qm.T, Q.T, ib.T)
    bpw = B // _NW
    Gu = _sc_gather(u.reshape(_NW, bpw // _CH, _CH), Cu)
    Gi = _sc_gather(i.reshape(_NW, bpw // _CH, _CH), Ci)
    return _tc_mlp(Gu, Gi, W1, b1, W2, b2, Wo, bo)


# speedup vs baseline: 5.8496x; 1.2520x over previous
import functools

import jax
import jax.numpy as jnp
from jax import lax
from jax.experimental import pallas as pl
from jax.experimental.pallas import tpu as pltpu
from jax.experimental.pallas import tpu_sc as plsc

_NC = 2
_NS = 16
_NW = _NC * _NS
_CH = 128
_W = 128
_RB = 4096
_DMLP = 64
_DM = 32
_BCOL = _DMLP // 2 + _DM // 2


def _tc_pack(Pmt, Pt, ubt, Qmt, Qt, ibt):
    n = Pmt.shape[1]
    nblk = (n + _RB - 1) // _RB
    dall = _DMLP + _DM + 1
    hb = _RB // 2

    def pack_side(e_ref, g_ref, b_ref, o_ref, sel_lo, sel_hi, bmask):
        x = jnp.concatenate([e_ref[...], g_ref[...], b_ref[...]], axis=0)
        t_lo = lax.dot_general(x, sel_lo, (((0,), (0,)), ((), ())),
                               preferred_element_type=jnp.float32)
        t_hi = lax.dot_general(x, sel_hi, (((0,), (0,)), ((), ())),
                               preferred_element_type=jnp.float32)
        half = jnp.uint32(0x8000)
        hmask = jnp.uint32(0xFFFF0000)
        blo = lax.bitcast_convert_type(t_lo, jnp.uint32)
        bhi = lax.bitcast_convert_type(t_hi, jnp.uint32)
        packed = lax.bitcast_convert_type(
            ((blo + half) >> 16) | ((bhi + half) & hmask), jnp.float32)
        row = jnp.where(bmask, t_lo, packed)
        o_ref[...] = jnp.concatenate([row[0:hb], row[hb:_RB]], axis=1)

    def body(pm_ref, p_ref, ub_ref, qm_ref, q_ref, ib_ref, ou_ref, oi_ref):
        r = lax.broadcasted_iota(jnp.int32, (dall, 64), 0)
        c = lax.broadcasted_iota(jnp.int32, (dall, 64), 1)
        sel_lo = ((r == c) & (c < 32)
                  | (r - 32 == c) & (c >= 32) & (c < 48)
                  | (r == 96) & (c == 48)).astype(jnp.float32)
        sel_hi = ((r - 32 == c) & (c < 32)
                  | (r - 48 == c) & (c >= 32) & (c < 48)).astype(jnp.float32)
        bmask = lax.broadcasted_iota(jnp.int32, (_RB, 64), 1) == _BCOL
        pack_side(pm_ref, p_ref, ub_ref, ou_ref, sel_lo, sel_hi, bmask)
        pack_side(qm_ref, q_ref, ib_ref, oi_ref, sel_lo, sel_hi, bmask)

    out = jax.ShapeDtypeStruct((nblk * hb, _W), jnp.float32)
    return pl.pallas_call(
        body,
        grid=(nblk,),
        in_specs=[
            pl.BlockSpec((_DMLP, _RB), lambda b: (0, b)),
            pl.BlockSpec((_DM, _RB), lambda b: (0, b)),
            pl.BlockSpec((1, _RB), lambda b: (0, b)),
            pl.BlockSpec((_DMLP, _RB), lambda b: (0, b)),
            pl.BlockSpec((_DM, _RB), lambda b: (0, b)),
            pl.BlockSpec((1, _RB), lambda b: (0, b)),
        ],
        out_specs=[
            pl.BlockSpec((hb, _W), lambda b: (b, 0)),
            pl.BlockSpec((hb, _W), lambda b: (b, 0)),
        ],
        out_shape=[out, out],
    )(Pmt, Pt, ubt, Qmt, Qt, ibt)


def _sc_gather(idx3, C):
    nw, nch, ch = idx3.shape
    bpw = nch * ch
    n2 = nw * bpw
    cpb = 2
    stage = cpb * ch
    nstg = nch // cpb
    mesh = plsc.VectorSubcoreMesh(core_axis_name="c", subcore_axis_name="s")

    @functools.partial(
        pl.kernel,
        mesh=mesh,
        out_type=jax.ShapeDtypeStruct((n2, _W), jnp.float32),
        scratch_types=[
            pltpu.VMEM((nch, ch), jnp.int32),
            pltpu.VMEM((stage, _W), jnp.float32),
            pltpu.VMEM((stage, _W), jnp.float32),
            pltpu.SemaphoreType.DMA,
            pltpu.SemaphoreType.DMA,
            pltpu.SemaphoreType.DMA,
            pltpu.SemaphoreType.DMA,
        ],
    )
    def k(idx_hbm, c_hbm, g_o, iv, rows0, rows1, gsem0, gsem1, osem0, osem1):
        wid = lax.axis_index("s") * _NC + lax.axis_index("c")
        base = wid * bpw
        pltpu.sync_copy(idx_hbm.at[wid], iv)
        bufs = (rows0, rows1)
        gsems = (gsem0, gsem1)
        osems = (osem0, osem1)

        def gathers(s):
            b = s % 2
            return [
                pltpu.make_async_copy(
                    c_hbm.at[iv.at[s * cpb + j]],
                    bufs[b].at[pl.ds(j * ch, ch)], gsems[b])
                for j in range(cpb)
            ]

        def copy_out(s):
            b = s % 2
            return pltpu.make_async_copy(
                bufs[b], g_o.at[pl.ds(base + s * stage, stage)], osems[b])

        for c in gathers(0):
            c.start()
        for s in range(nstg):
            for c in gathers(s):
                c.wait()
            if s + 1 < nstg:
                if s >= 1:
                    copy_out(s - 1).wait()
                for c in gathers(s + 1):
                    c.start()
            copy_out(s).start()
        copy_out(nstg - 1).wait()

    return k(idx3, C)


def _tc_mlp(Gu, Gi, hu, hi, W1, b1, W2, b2, Wo, bo):
    B = Gu.shape[0]
    BLK = 2048
    h1d = W1.shape[1]
    h2d = W2.shape[1]

    def body(gu_ref, gi_ref, hu_ref, hi_ref,
             w1_ref, b1_ref, w2_ref, b2_ref, wo_ref, bo_ref, o_ref):
        f32 = jnp.float32

        def unpack(g_ref, h_ref):
            g = g_ref[...]
            h = h_ref[...]
            sel = jnp.where(h[:, None] > 0.5, g[:, 64:128], g[:, 0:64])
            bias = sel[:, _BCOL]
            uw = lax.bitcast_convert_type(sel, jnp.uint32)
            lo = lax.bitcast_convert_type(uw << 16, f32)
            hi = lax.bitcast_convert_type(uw & jnp.uint32(0xFFFF0000), f32)
            pmu = jnp.concatenate([lo[:, 0:32], hi[:, 0:32]], axis=1)
            pu = jnp.concatenate([lo[:, 32:48], hi[:, 32:48]], axis=1)
            return pmu, pu, bias

        pmu, pu, ubb = unpack(gu_ref, hu_ref)
        qmi, qi, ibb = unpack(gi_ref, hi_ref)
        h1 = jnp.dot(pmu, w1_ref[0:_DMLP, :], preferred_element_type=f32)
        h1 += jnp.dot(qmi, w1_ref[_DMLP:2 * _DMLP, :], preferred_element_type=f32)
        h1 = jnp.maximum(h1 + b1_ref[...], 0.0)
        h2 = jnp.dot(h1, w2_ref[...], preferred_element_type=f32) + b2_ref[...]
        h2 = jnp.maximum(h2, 0.0)
        gmf = pu * qi
        s = jnp.dot(h2, wo_ref[0:h2d, :], preferred_element_type=f32)
        s += jnp.dot(gmf, wo_ref[h2d:h2d + _DM, :], preferred_element_type=f32)
        o_ref[...] = s[:, 0] + bo_ref[0] + ubb + ibb

    grid = (B // BLK,)
    return pl.pallas_call(
        body,
        grid=grid,
        in_specs=[
            pl.BlockSpec((BLK, _W), lambda b: (b, 0)),
            pl.BlockSpec((BLK, _W), lambda b: (b, 0)),
            pl.BlockSpec((BLK,), lambda b: (b,)),
            pl.BlockSpec((BLK,), lambda b: (b,)),
            pl.BlockSpec((2 * _DMLP, h1d), lambda b: (0, 0)),
            pl.BlockSpec((h1d,), lambda b: (0,)),
            pl.BlockSpec((h1d, h2d), lambda b: (0, 0)),
            pl.BlockSpec((h2d,), lambda b: (0,)),
            pl.BlockSpec((h2d + _DM, 1), lambda b: (0, 0)),
            pl.BlockSpec((1,), lambda b: (0,)),
        ],
        out_specs=pl.BlockSpec((BLK,), lambda b: (b,)),
        out_shape=jax.ShapeDtypeStruct((B,), jnp.float32),
    )(Gu, Gi, hu, hi, W1, b1, W2, b2, Wo, bo)


def _rows_halves(idx):
    hb = _RB // 2
    rows = (idx // _RB) * hb + (idx % hb)
    halves = ((idx // hb) % 2).astype(jnp.float32)
    return rows, halves


def kernel(u, i, P, Q, Pm, Qm, W1, b1, W2, b2, Wo, bo, ub, ib):
    B = u.shape[0]
    u = u.astype(jnp.int32)
    i = i.astype(jnp.int32)
    Cu, Ci = _tc_pack(Pm.T, P.T, ub.T, Qm.T, Q.T, ib.T)
    ru, hu = _rows_halves(u)
    ri, hi = _rows_halves(i)
    bpw = B // _NW
    Gu = _sc_gather(ru.reshape(_NW, bpw // _CH, _CH), Cu)
    Gi = _sc_gather(ri.reshape(_NW, bpw // _CH, _CH), Ci)
    return _tc_mlp(Gu, Gi, hu, hi, W1, b1, W2, b2, Wo, bo)


# speedup vs baseline: 6.6552x; 1.1377x over previous
import functools

import jax
import jax.numpy as jnp
from jax import lax
from jax.experimental import pallas as pl
from jax.experimental.pallas import tpu as pltpu
from jax.experimental.pallas import tpu_sc as plsc

_NC = 2
_NS = 16
_NW = _NC * _NS
_CH = 128
_W = 128
_RB = 8192
_DMLP = 64
_DM = 32
_BCOL = _DMLP // 2 + _DM // 2


def _tc_pack(Pmt, Pt, ubt, Qmt, Qt, ibt):
    n = Pmt.shape[1]
    nblk = (n + _RB - 1) // _RB
    dall = _DMLP + _DM + 1
    hb = _RB // 2

    def pack_side(e_ref, g_ref, b_ref, o_ref, sel_lo, sel_hi, bmask):
        x = jnp.concatenate([e_ref[...], g_ref[...], b_ref[...]], axis=0)
        t_lo = lax.dot_general(x, sel_lo, (((0,), (0,)), ((), ())),
                               preferred_element_type=jnp.float32)
        t_hi = lax.dot_general(x, sel_hi, (((0,), (0,)), ((), ())),
                               preferred_element_type=jnp.float32)
        half = jnp.uint32(0x8000)
        hmask = jnp.uint32(0xFFFF0000)
        blo = lax.bitcast_convert_type(t_lo, jnp.uint32)
        bhi = lax.bitcast_convert_type(t_hi, jnp.uint32)
        packed = lax.bitcast_convert_type(
            ((blo + half) >> 16) | ((bhi + half) & hmask), jnp.float32)
        row = jnp.where(bmask, t_lo, packed)
        o_ref[...] = jnp.concatenate([row[0:hb], row[hb:_RB]], axis=1)

    def body(pm_ref, p_ref, ub_ref, qm_ref, q_ref, ib_ref, ou_ref, oi_ref):
        r = lax.broadcasted_iota(jnp.int32, (dall, 64), 0)
        c = lax.broadcasted_iota(jnp.int32, (dall, 64), 1)
        sel_lo = ((r == c) & (c < 32)
                  | (r - 32 == c) & (c >= 32) & (c < 48)
                  | (r == 96) & (c == 48)).astype(jnp.float32)
        sel_hi = ((r - 32 == c) & (c < 32)
                  | (r - 48 == c) & (c >= 32) & (c < 48)).astype(jnp.float32)
        bmask = lax.broadcasted_iota(jnp.int32, (_RB, 64), 1) == _BCOL
        pack_side(pm_ref, p_ref, ub_ref, ou_ref, sel_lo, sel_hi, bmask)
        pack_side(qm_ref, q_ref, ib_ref, oi_ref, sel_lo, sel_hi, bmask)

    out = jax.ShapeDtypeStruct((nblk * hb, _W), jnp.float32)
    return pl.pallas_call(
        body,
        grid=(nblk,),
        in_specs=[
            pl.BlockSpec((_DMLP, _RB), lambda b: (0, b)),
            pl.BlockSpec((_DM, _RB), lambda b: (0, b)),
            pl.BlockSpec((1, _RB), lambda b: (0, b)),
            pl.BlockSpec((_DMLP, _RB), lambda b: (0, b)),
            pl.BlockSpec((_DM, _RB), lambda b: (0, b)),
            pl.BlockSpec((1, _RB), lambda b: (0, b)),
        ],
        out_specs=[
            pl.BlockSpec((hb, _W), lambda b: (b, 0)),
            pl.BlockSpec((hb, _W), lambda b: (b, 0)),
        ],
        out_shape=[out, out],
    )(Pmt, Pt, ubt, Qmt, Qt, ibt)


def _sc_gather(idx3, C):
    nw, nch, ch = idx3.shape
    bpw = nch * ch
    n2 = nw * bpw
    cpb = 2
    stage = cpb * ch
    nstg = nch // cpb
    mesh = plsc.VectorSubcoreMesh(core_axis_name="c", subcore_axis_name="s")

    @functools.partial(
        pl.kernel,
        mesh=mesh,
        out_type=jax.ShapeDtypeStruct((n2, _W), jnp.float32),
        scratch_types=[
            pltpu.VMEM((nch, ch), jnp.int32),
            pltpu.VMEM((stage, _W), jnp.float32),
            pltpu.VMEM((stage, _W), jnp.float32),
            pltpu.SemaphoreType.DMA,
            pltpu.SemaphoreType.DMA,
            pltpu.SemaphoreType.DMA,
            pltpu.SemaphoreType.DMA,
        ],
    )
    def k(idx_hbm, c_hbm, g_o, iv, rows0, rows1, gsem0, gsem1, osem0, osem1):
        wid = lax.axis_index("s") * _NC + lax.axis_index("c")
        base = wid * bpw
        pltpu.sync_copy(idx_hbm.at[wid], iv)
        bufs = (rows0, rows1)
        gsems = (gsem0, gsem1)
        osems = (osem0, osem1)

        def gathers(s):
            b = s % 2
            return [
                pltpu.make_async_copy(
                    c_hbm.at[iv.at[s * cpb + j]],
                    bufs[b].at[pl.ds(j * ch, ch)], gsems[b])
                for j in range(cpb)
            ]

        def copy_out(s):
            b = s % 2
            return pltpu.make_async_copy(
                bufs[b], g_o.at[pl.ds(base + s * stage, stage)], osems[b])

        for c in gathers(0):
            c.start()
        for s in range(nstg):
            for c in gathers(s):
                c.wait()
            if s + 1 < nstg:
                if s >= 1:
                    copy_out(s - 1).wait()
                for c in gathers(s + 1):
                    c.start()
            copy_out(s).start()
        copy_out(nstg - 1).wait()

    return k(idx3, C)


def _tc_mlp(Gu, Gi, hu, hi, W1, b1, W2, b2, Wo, bo):
    B = Gu.shape[0]
    BLK = 2048
    h1d = W1.shape[1]
    h2d = W2.shape[1]

    def body(gu_ref, gi_ref, hu_ref, hi_ref,
             w1_ref, b1_ref, w2_ref, b2_ref, wo_ref, bo_ref, o_ref):
        f32 = jnp.float32

        def unpack(g_ref, h_ref):
            g = g_ref[...]
            h = h_ref[...]
            sel = jnp.where(h[:, None] > 0.5, g[:, 64:128], g[:, 0:64])
            bias = sel[:, _BCOL]
            uw = lax.bitcast_convert_type(sel, jnp.uint32)
            lo = lax.bitcast_convert_type(uw << 16, f32)
            hi = lax.bitcast_convert_type(uw & jnp.uint32(0xFFFF0000), f32)
            pmu = jnp.concatenate([lo[:, 0:32], hi[:, 0:32]], axis=1)
            pu = jnp.concatenate([lo[:, 32:48], hi[:, 32:48]], axis=1)
            return pmu, pu, bias

        pmu, pu, ubb = unpack(gu_ref, hu_ref)
        qmi, qi, ibb = unpack(gi_ref, hi_ref)
        h1 = jnp.dot(pmu, w1_ref[0:_DMLP, :], preferred_element_type=f32)
        h1 += jnp.dot(qmi, w1_ref[_DMLP:2 * _DMLP, :], preferred_element_type=f32)
        h1 = jnp.maximum(h1 + b1_ref[...], 0.0)
        h2 = jnp.dot(h1, w2_ref[...], preferred_element_type=f32) + b2_ref[...]
        h2 = jnp.maximum(h2, 0.0)
        gmf = pu * qi
        s = jnp.dot(h2, wo_ref[0:h2d, :], preferred_element_type=f32)
        s += jnp.dot(gmf, wo_ref[h2d:h2d + _DM, :], preferred_element_type=f32)
        o_ref[...] = s[:, 0] + bo_ref[0] + ubb + ibb

    grid = (B // BLK,)
    return pl.pallas_call(
        body,
        grid=grid,
        in_specs=[
            pl.BlockSpec((BLK, _W), lambda b: (b, 0)),
            pl.BlockSpec((BLK, _W), lambda b: (b, 0)),
            pl.BlockSpec((BLK,), lambda b: (b,)),
            pl.BlockSpec((BLK,), lambda b: (b,)),
            pl.BlockSpec((2 * _DMLP, h1d), lambda b: (0, 0)),
            pl.BlockSpec((h1d,), lambda b: (0,)),
            pl.BlockSpec((h1d, h2d), lambda b: (0, 0)),
            pl.BlockSpec((h2d,), lambda b: (0,)),
            pl.BlockSpec((h2d + _DM, 1), lambda b: (0, 0)),
            pl.BlockSpec((1,), lambda b: (0,)),
        ],
        out_specs=pl.BlockSpec((BLK,), lambda b: (b,)),
        out_shape=jax.ShapeDtypeStruct((B,), jnp.float32),
    )(Gu, Gi, hu, hi, W1, b1, W2, b2, Wo, bo)


def _rows_halves(idx):
    hb = _RB // 2
    rows = (idx // _RB) * hb + (idx % hb)
    halves = ((idx // hb) % 2).astype(jnp.float32)
    return rows, halves


def kernel(u, i, P, Q, Pm, Qm, W1, b1, W2, b2, Wo, bo, ub, ib):
    B = u.shape[0]
    u = u.astype(jnp.int32)
    i = i.astype(jnp.int32)
    Cu, Ci = _tc_pack(Pm.T, P.T, ub.T, Qm.T, Q.T, ib.T)
    ru, hu = _rows_halves(u)
    ri, hi = _rows_halves(i)
    bpw = B // _NW
    Gu = _sc_gather(ru.reshape(_NW, bpw // _CH, _CH), Cu)
    Gi = _sc_gather(ri.reshape(_NW, bpw // _CH, _CH), Ci)
    return _tc_mlp(Gu, Gi, hu, hi, W1, b1, W2, b2, Wo, bo)


# speedup vs baseline: 7.1018x; 1.0671x over previous
import functools

import jax
import jax.numpy as jnp
from jax import lax
from jax.experimental import pallas as pl
from jax.experimental.pallas import tpu as pltpu
from jax.experimental.pallas import tpu_sc as plsc

_NC = 2
_NS = 16
_NW = _NC * _NS
_CH = 128
_W = 128
_RB = 16384
_DMLP = 64
_DM = 32
_BCOL = _DMLP // 2 + _DM // 2


def _tc_pack(Pmt, Pt, ubt, Qmt, Qt, ibt):
    n = Pmt.shape[1]
    nblk = (n + _RB - 1) // _RB
    dall = _DMLP + _DM + 1
    hb = _RB // 2

    def pack_side(e_ref, g_ref, b_ref, o_ref, sel_lo, sel_hi, bmask):
        x = jnp.concatenate([e_ref[...], g_ref[...], b_ref[...]], axis=0)
        t_lo = lax.dot_general(x, sel_lo, (((0,), (0,)), ((), ())),
                               preferred_element_type=jnp.float32)
        t_hi = lax.dot_general(x, sel_hi, (((0,), (0,)), ((), ())),
                               preferred_element_type=jnp.float32)
        half = jnp.uint32(0x8000)
        hmask = jnp.uint32(0xFFFF0000)
        blo = lax.bitcast_convert_type(t_lo, jnp.uint32)
        bhi = lax.bitcast_convert_type(t_hi, jnp.uint32)
        packed = lax.bitcast_convert_type(
            ((blo + half) >> 16) | ((bhi + half) & hmask), jnp.float32)
        row = jnp.where(bmask, t_lo, packed)
        o_ref[...] = jnp.concatenate([row[0:hb], row[hb:_RB]], axis=1)

    def body(pm_ref, p_ref, ub_ref, qm_ref, q_ref, ib_ref, ou_ref, oi_ref):
        r = lax.broadcasted_iota(jnp.int32, (dall, 64), 0)
        c = lax.broadcasted_iota(jnp.int32, (dall, 64), 1)
        sel_lo = ((r == c) & (c < 32)
                  | (r - 32 == c) & (c >= 32) & (c < 48)
                  | (r == 96) & (c == 48)).astype(jnp.float32)
        sel_hi = ((r - 32 == c) & (c < 32)
                  | (r - 48 == c) & (c >= 32) & (c < 48)).astype(jnp.float32)
        bmask = lax.broadcasted_iota(jnp.int32, (_RB, 64), 1) == _BCOL
        pack_side(pm_ref, p_ref, ub_ref, ou_ref, sel_lo, sel_hi, bmask)
        pack_side(qm_ref, q_ref, ib_ref, oi_ref, sel_lo, sel_hi, bmask)

    out = jax.ShapeDtypeStruct((nblk * hb, _W), jnp.float32)
    return pl.pallas_call(
        body,
        grid=(nblk,),
        in_specs=[
            pl.BlockSpec((_DMLP, _RB), lambda b: (0, b)),
            pl.BlockSpec((_DM, _RB), lambda b: (0, b)),
            pl.BlockSpec((1, _RB), lambda b: (0, b)),
            pl.BlockSpec((_DMLP, _RB), lambda b: (0, b)),
            pl.BlockSpec((_DM, _RB), lambda b: (0, b)),
            pl.BlockSpec((1, _RB), lambda b: (0, b)),
        ],
        out_specs=[
            pl.BlockSpec((hb, _W), lambda b: (b, 0)),
            pl.BlockSpec((hb, _W), lambda b: (b, 0)),
        ],
        out_shape=[out, out],
    )(Pmt, Pt, ubt, Qmt, Qt, ibt)


def _sc_gather(idx3, C):
    nw, nch, ch = idx3.shape
    bpw = nch * ch
    n2 = nw * bpw
    cpb = 2
    stage = cpb * ch
    nstg = nch // cpb
    mesh = plsc.VectorSubcoreMesh(core_axis_name="c", subcore_axis_name="s")

    @functools.partial(
        pl.kernel,
        mesh=mesh,
        out_type=jax.ShapeDtypeStruct((n2, _W), jnp.float32),
        scratch_types=[
            pltpu.VMEM((nch, ch), jnp.int32),
            pltpu.VMEM((stage, _W), jnp.float32),
            pltpu.VMEM((stage, _W), jnp.float32),
            pltpu.SemaphoreType.DMA,
            pltpu.SemaphoreType.DMA,
            pltpu.SemaphoreType.DMA,
            pltpu.SemaphoreType.DMA,
        ],
    )
    def k(idx_hbm, c_hbm, g_o, iv, rows0, rows1, gsem0, gsem1, osem0, osem1):
        wid = lax.axis_index("s") * _NC + lax.axis_index("c")
        base = wid * bpw
        pltpu.sync_copy(idx_hbm.at[wid], iv)
        bufs = (rows0, rows1)
        gsems = (gsem0, gsem1)
        osems = (osem0, osem1)

        def gathers(s):
            b = s % 2
            return [
                pltpu.make_async_copy(
                    c_hbm.at[iv.at[s * cpb + j]],
                    bufs[b].at[pl.ds(j * ch, ch)], gsems[b])
                for j in range(cpb)
            ]

        def copy_out(s):
            b = s % 2
            return pltpu.make_async_copy(
                bufs[b], g_o.at[pl.ds(base + s * stage, stage)], osems[b])

        for c in gathers(0):
            c.start()
        for s in range(nstg):
            for c in gathers(s):
                c.wait()
            if s + 1 < nstg:
                if s >= 1:
                    copy_out(s - 1).wait()
                for c in gathers(s + 1):
                    c.start()
            copy_out(s).start()
        copy_out(nstg - 1).wait()

    return k(idx3, C)


def _tc_mlp(Gu, Gi, hu, hi, W1, b1, W2, b2, Wo, bo):
    B = Gu.shape[0]
    BLK = 2048
    h1d = W1.shape[1]
    h2d = W2.shape[1]

    def body(gu_ref, gi_ref, hu_ref, hi_ref,
             w1_ref, b1_ref, w2_ref, b2_ref, wo_ref, bo_ref, o_ref):
        f32 = jnp.float32

        def unpack(g_ref, h_ref):
            g = g_ref[...]
            h = h_ref[...]
            sel = jnp.where(h[:, None] > 0.5, g[:, 64:128], g[:, 0:64])
            bias = sel[:, _BCOL]
            uw = lax.bitcast_convert_type(sel, jnp.uint32)
            lo = lax.bitcast_convert_type(uw << 16, f32)
            hi = lax.bitcast_convert_type(uw & jnp.uint32(0xFFFF0000), f32)
            pmu = jnp.concatenate([lo[:, 0:32], hi[:, 0:32]], axis=1)
            pu = jnp.concatenate([lo[:, 32:48], hi[:, 32:48]], axis=1)
            return pmu, pu, bias

        pmu, pu, ubb = unpack(gu_ref, hu_ref)
        qmi, qi, ibb = unpack(gi_ref, hi_ref)
        h1 = jnp.dot(pmu, w1_ref[0:_DMLP, :], preferred_element_type=f32)
        h1 += jnp.dot(qmi, w1_ref[_DMLP:2 * _DMLP, :], preferred_element_type=f32)
        h1 = jnp.maximum(h1 + b1_ref[...], 0.0)
        h2 = jnp.dot(h1, w2_ref[...], preferred_element_type=f32) + b2_ref[...]
        h2 = jnp.maximum(h2, 0.0)
        gmf = pu * qi
        s = jnp.dot(h2, wo_ref[0:h2d, :], preferred_element_type=f32)
        s += jnp.dot(gmf, wo_ref[h2d:h2d + _DM, :], preferred_element_type=f32)
        o_ref[...] = s[:, 0] + bo_ref[0] + ubb + ibb

    grid = (B // BLK,)
    return pl.pallas_call(
        body,
        grid=grid,
        in_specs=[
            pl.BlockSpec((BLK, _W), lambda b: (b, 0)),
            pl.BlockSpec((BLK, _W), lambda b: (b, 0)),
            pl.BlockSpec((BLK,), lambda b: (b,)),
            pl.BlockSpec((BLK,), lambda b: (b,)),
            pl.BlockSpec((2 * _DMLP, h1d), lambda b: (0, 0)),
            pl.BlockSpec((h1d,), lambda b: (0,)),
            pl.BlockSpec((h1d, h2d), lambda b: (0, 0)),
            pl.BlockSpec((h2d,), lambda b: (0,)),
            pl.BlockSpec((h2d + _DM, 1), lambda b: (0, 0)),
            pl.BlockSpec((1,), lambda b: (0,)),
        ],
        out_specs=pl.BlockSpec((BLK,), lambda b: (b,)),
        out_shape=jax.ShapeDtypeStruct((B,), jnp.float32),
    )(Gu, Gi, hu, hi, W1, b1, W2, b2, Wo, bo)


def _rows_halves(idx):
    hb = _RB // 2
    rows = (idx // _RB) * hb + (idx % hb)
    halves = ((idx // hb) % 2).astype(jnp.float32)
    return rows, halves


def kernel(u, i, P, Q, Pm, Qm, W1, b1, W2, b2, Wo, bo, ub, ib):
    B = u.shape[0]
    u = u.astype(jnp.int32)
    i = i.astype(jnp.int32)
    Cu, Ci = _tc_pack(Pm.T, P.T, ub.T, Qm.T, Q.T, ib.T)
    ru, hu = _rows_halves(u)
    ri, hi = _rows_halves(i)
    bpw = B // _NW
    Gu = _sc_gather(ru.reshape(_NW, bpw // _CH, _CH), Cu)
    Gi = _sc_gather(ri.reshape(_NW, bpw // _CH, _CH), Ci)
    return _tc_mlp(Gu, Gi, hu, hi, W1, b1, W2, b2, Wo, bo)


# speedup vs baseline: 7.1914x; 1.0126x over previous
import functools

import jax
import jax.numpy as jnp
from jax import lax
from jax.experimental import pallas as pl
from jax.experimental.pallas import tpu as pltpu
from jax.experimental.pallas import tpu_sc as plsc

_NC = 2
_NS = 16
_NW = _NC * _NS
_CH = 128
_W = 128
_RB = 16384
_DMLP = 64
_DM = 32
_BCOL = _DMLP // 2 + _DM // 2


def _tc_pack(Pmt, Pt, ubt, Qmt, Qt, ibt):
    n = Pmt.shape[1]
    nblk = (n + _RB - 1) // _RB
    dall = _DMLP + _DM + 1
    hb = _RB // 2

    def pack_side(e_ref, g_ref, b_ref, o_ref, sel_lo, sel_hi, bmask):
        x = jnp.concatenate([e_ref[...], g_ref[...], b_ref[...]], axis=0)
        t_lo = lax.dot_general(x, sel_lo, (((0,), (0,)), ((), ())),
                               preferred_element_type=jnp.float32)
        t_hi = lax.dot_general(x, sel_hi, (((0,), (0,)), ((), ())),
                               preferred_element_type=jnp.float32)
        half = jnp.uint32(0x8000)
        hmask = jnp.uint32(0xFFFF0000)
        blo = lax.bitcast_convert_type(t_lo, jnp.uint32)
        bhi = lax.bitcast_convert_type(t_hi, jnp.uint32)
        packed = lax.bitcast_convert_type(
            ((blo + half) >> 16) | ((bhi + half) & hmask), jnp.float32)
        row = jnp.where(bmask, t_lo, packed)
        o_ref[...] = jnp.concatenate([row[0:hb], row[hb:_RB]], axis=1)

    def body(pm_ref, p_ref, ub_ref, qm_ref, q_ref, ib_ref, ou_ref, oi_ref):
        r = lax.broadcasted_iota(jnp.int32, (dall, 64), 0)
        c = lax.broadcasted_iota(jnp.int32, (dall, 64), 1)
        sel_lo = ((r == c) & (c < 32)
                  | (r - 32 == c) & (c >= 32) & (c < 48)
                  | (r == 96) & (c == 48)).astype(jnp.float32)
        sel_hi = ((r - 32 == c) & (c < 32)
                  | (r - 48 == c) & (c >= 32) & (c < 48)).astype(jnp.float32)
        bmask = lax.broadcasted_iota(jnp.int32, (_RB, 64), 1) == _BCOL
        pack_side(pm_ref, p_ref, ub_ref, ou_ref, sel_lo, sel_hi, bmask)
        pack_side(qm_ref, q_ref, ib_ref, oi_ref, sel_lo, sel_hi, bmask)

    out = jax.ShapeDtypeStruct((nblk * hb, _W), jnp.float32)
    return pl.pallas_call(
        body,
        grid=(nblk,),
        in_specs=[
            pl.BlockSpec((_DMLP, _RB), lambda b: (0, b)),
            pl.BlockSpec((_DM, _RB), lambda b: (0, b)),
            pl.BlockSpec((1, _RB), lambda b: (0, b)),
            pl.BlockSpec((_DMLP, _RB), lambda b: (0, b)),
            pl.BlockSpec((_DM, _RB), lambda b: (0, b)),
            pl.BlockSpec((1, _RB), lambda b: (0, b)),
        ],
        out_specs=[
            pl.BlockSpec((hb, _W), lambda b: (b, 0)),
            pl.BlockSpec((hb, _W), lambda b: (b, 0)),
        ],
        out_shape=[out, out],
    )(Pmt, Pt, ubt, Qmt, Qt, ibt)


def _sc_gather(iu3, ii3, Cu, Ci):
    nw, nch, ch = iu3.shape
    bpw = nch * ch
    n2 = nw * bpw
    cpb = 2
    stage = cpb * ch
    nstg = nch // cpb
    out = jax.ShapeDtypeStruct((n2, _W), jnp.float32)
    mesh = plsc.VectorSubcoreMesh(core_axis_name="c", subcore_axis_name="s")

    @functools.partial(
        pl.kernel,
        mesh=mesh,
        out_type=[out, out],
        scratch_types=[
            pltpu.VMEM((nch, ch), jnp.int32),
            pltpu.VMEM((nch, ch), jnp.int32),
            pltpu.VMEM((stage, _W), jnp.float32),
            pltpu.VMEM((stage, _W), jnp.float32),
            pltpu.SemaphoreType.DMA,
            pltpu.SemaphoreType.DMA,
            pltpu.SemaphoreType.DMA,
            pltpu.SemaphoreType.DMA,
        ],
    )
    def k(iu_hbm, ii_hbm, cu_hbm, ci_hbm, gu_o, gi_o,
          iuv, iiv, rows0, rows1, gsem0, gsem1, osem0, osem1):
        wid = lax.axis_index("s") * _NC + lax.axis_index("c")
        base = wid * bpw
        pltpu.sync_copy(iu_hbm.at[wid], iuv)
        pltpu.sync_copy(ii_hbm.at[wid], iiv)
        bufs = (rows0, rows1)
        gsems = (gsem0, gsem1)
        osems = (osem0, osem1)
        nst2 = 2 * nstg

        def side(s):
            return (iuv, cu_hbm, gu_o, s) if s < nstg else (
                iiv, ci_hbm, gi_o, s - nstg)

        def gathers(s):
            b = s % 2
            iv, c_hbm, _, sl = side(s)
            return [
                pltpu.make_async_copy(
                    c_hbm.at[iv.at[sl * cpb + j]],
                    bufs[b].at[pl.ds(j * ch, ch)], gsems[b])
                for j in range(cpb)
            ]

        def copy_out(s):
            b = s % 2
            _, _, g_o, sl = side(s)
            return pltpu.make_async_copy(
                bufs[b], g_o.at[pl.ds(base + sl * stage, stage)], osems[b])

        for c in gathers(0):
            c.start()
        for s in range(nst2):
            for c in gathers(s):
                c.wait()
            if s + 1 < nst2:
                if s >= 1:
                    copy_out(s - 1).wait()
                for c in gathers(s + 1):
                    c.start()
            copy_out(s).start()
        copy_out(nst2 - 1).wait()

    return k(iu3, ii3, Cu, Ci)


def _tc_mlp(Gu, Gi, hu, hi, W1, b1, W2, b2, Wo, bo):
    B = Gu.shape[0]
    BLK = 2048
    h1d = W1.shape[1]
    h2d = W2.shape[1]

    def body(gu_ref, gi_ref, hu_ref, hi_ref,
             w1_ref, b1_ref, w2_ref, b2_ref, wo_ref, bo_ref, o_ref):
        f32 = jnp.float32

        def unpack(g_ref, h_ref):
            g = g_ref[...]
            h = h_ref[...]
            sel = jnp.where(h[:, None] > 0.5, g[:, 64:128], g[:, 0:64])
            bias = sel[:, _BCOL]
            uw = lax.bitcast_convert_type(sel, jnp.uint32)
            lo = lax.bitcast_convert_type(uw << 16, f32)
            hi = lax.bitcast_convert_type(uw & jnp.uint32(0xFFFF0000), f32)
            pmu = jnp.concatenate([lo[:, 0:32], hi[:, 0:32]], axis=1)
            pu = jnp.concatenate([lo[:, 32:48], hi[:, 32:48]], axis=1)
            return pmu, pu, bias

        pmu, pu, ubb = unpack(gu_ref, hu_ref)
        qmi, qi, ibb = unpack(gi_ref, hi_ref)
        h1 = jnp.dot(pmu, w1_ref[0:_DMLP, :], preferred_element_type=f32)
        h1 += jnp.dot(qmi, w1_ref[_DMLP:2 * _DMLP, :], preferred_element_type=f32)
        h1 = jnp.maximum(h1 + b1_ref[...], 0.0)
        h2 = jnp.dot(h1, w2_ref[...], preferred_element_type=f32) + b2_ref[...]
        h2 = jnp.maximum(h2, 0.0)
        gmf = pu * qi
        s = jnp.dot(h2, wo_ref[0:h2d, :], preferred_element_type=f32)
        s += jnp.dot(gmf, wo_ref[h2d:h2d + _DM, :], preferred_element_type=f32)
        o_ref[...] = s[:, 0] + bo_ref[0] + ubb + ibb

    grid = (B // BLK,)
    return pl.pallas_call(
        body,
        grid=grid,
        in_specs=[
            pl.BlockSpec((BLK, _W), lambda b: (b, 0)),
            pl.BlockSpec((BLK, _W), lambda b: (b, 0)),
            pl.BlockSpec((BLK,), lambda b: (b,)),
            pl.BlockSpec((BLK,), lambda b: (b,)),
            pl.BlockSpec((2 * _DMLP, h1d), lambda b: (0, 0)),
            pl.BlockSpec((h1d,), lambda b: (0,)),
            pl.BlockSpec((h1d, h2d), lambda b: (0, 0)),
            pl.BlockSpec((h2d,), lambda b: (0,)),
            pl.BlockSpec((h2d + _DM, 1), lambda b: (0, 0)),
            pl.BlockSpec((1,), lambda b: (0,)),
        ],
        out_specs=pl.BlockSpec((BLK,), lambda b: (b,)),
        out_shape=jax.ShapeDtypeStruct((B,), jnp.float32),
    )(Gu, Gi, hu, hi, W1, b1, W2, b2, Wo, bo)


def _rows_halves(idx):
    hb = _RB // 2
    rows = (idx // _RB) * hb + (idx % hb)
    halves = ((idx // hb) % 2).astype(jnp.float32)
    return rows, halves


def kernel(u, i, P, Q, Pm, Qm, W1, b1, W2, b2, Wo, bo, ub, ib):
    B = u.shape[0]
    u = u.astype(jnp.int32)
    i = i.astype(jnp.int32)
    Cu, Ci = _tc_pack(Pm.T, P.T, ub.T, Qm.T, Q.T, ib.T)
    ru, hu = _rows_halves(u)
    ri, hi = _rows_halves(i)
    bpw = B // _NW
    Gu, Gi = _sc_gather(ru.reshape(_NW, bpw // _CH, _CH),
                        ri.reshape(_NW, bpw // _CH, _CH), Cu, Ci)
    return _tc_mlp(Gu, Gi, hu, hi, W1, b1, W2, b2, Wo, bo)
